# baseline (speedup 1.0000x reference)
import functools

import jax
import jax.numpy as jnp
from jax import lax
from jax.experimental import pallas as pl
from jax.experimental.pallas import tpu as pltpu
from jax.experimental.pallas import tpu_sc as plsc

_V = 100000
_B = 1024
_EPS = 0.1 / (_V - 2)
_CONF = 0.9
_NW = 32
_BPW = _B // _NW
_L = 16
_BR = 64
_NUNITS = _V // _BR
_TAIL = _V - _NUNITS * _BR
_UPW = (_NUNITS + 1 + _NW - 1) // _NW
_NSLOT = 40


def _lanes():
    return lax.broadcasted_iota(jnp.int32, (_L,), 0)


def _unit_copies(out_hbm, u, tmpl, first, sem):
    off = pl.multiple_of(u * _BR, _BR)
    c_first = [
        pltpu.make_async_copy(first, out_hbm.at[pl.ds(0, 8)], sem),
        pltpu.make_async_copy(
            tmpl.at[pl.ds(0, _BR - 8)], out_hbm.at[pl.ds(8, _BR - 8)], sem),
    ]
    c_mid = [pltpu.make_async_copy(tmpl, out_hbm.at[pl.ds(off, _BR)], sem)]
    c_tail = [pltpu.make_async_copy(
        tmpl.at[pl.ds(0, _TAIL)],
        out_hbm.at[pl.ds(pl.multiple_of(_NUNITS * _BR + 0 * u, 8), _TAIL)],
        sem)]
    return c_first, c_mid, c_tail


def _for_units(wid, out_hbm, tmpl, first, sem, action, lo=0, hi=_UPW):
    def body(k, carry):
        u = wid + k * _NW
        c_first, c_mid, c_tail = _unit_copies(out_hbm, u, tmpl, first, sem)

        @pl.when(u == 0)
        def _():
            for cp in c_first:
                action(cp)

        @pl.when((u > 0) & (u < _NUNITS))
        def _():
            for cp in c_mid:
                action(cp)

        @pl.when(u == _NUNITS)
        def _():
            for cp in c_tail:
                action(cp)

        return carry

    lax.fori_loop(lo, hi, body, 0)


def _sc_body(trg_hbm, out_hbm, tmpl, first, patch, trg_v, sem_u, sem_p):
    wid = lax.axis_index("s") * 2 + lax.axis_index("c")

    pltpu.sync_copy(trg_hbm, trg_v)

    lanes = _lanes()
    zeros = jnp.zeros((_L,), jnp.float32)

    def tbody(c, carry):
        cb = pl.multiple_of(c * _L, _L)
        tv = trg_v[pl.ds(cb, _L)]
        vec = jnp.where(tv == 0, 0.0, _EPS).astype(jnp.float32)
        for i in range(_BR):
            tmpl[i, pl.ds(cb, _L)] = vec
        first[0, pl.ds(cb, _L)] = zeros
        for i in range(1, 8):
            first[i, pl.ds(cb, _L)] = vec
        return carry

    lax.fori_loop(0, _B // _L, tbody, 0)

    _for_units(wid, out_hbm, tmpl, first, sem_u, lambda cp: cp.start(),
               lo=0, hi=8)

    def scan(fn, lo=jnp.int32(0)):
        def body(c, n):
            vec = trg_v[pl.ds(c * _L, _L)]
            k = n
            for l in range(_L):
                t = vec[l]
                cond = (t != 0) & (jnp.bitwise_and(
                    lax.shift_right_logical(t, 6), _NW - 1) == wid)

                @pl.when(cond)
                def _(t=t, k=k, c=c, l=l):
                    fn(t, c * _L + l, k)

                k = jnp.where(cond, k + 1, k)
            return k

        return lax.fori_loop(0, _B // _L, body, lo)

    def compute_patch(t, b, slot):
        v8 = jnp.bitwise_and(t, -8)
        bcol = jnp.bitwise_and(b, -128)

        def pbody(cc, carry):
            tw = trg_v[pl.ds(bcol + cc * _L, _L)]
            colpad = tw == 0
            for i in range(8):
                v = v8 + i
                val = jnp.where(tw == v, _CONF, _EPS).astype(jnp.float32)
                val = jnp.where(colpad, 0.0, val)
                val = jnp.where(v == 0, 0.0, val)
                patch[slot, i, pl.ds(cc * _L, _L)] = val
            return carry

        lax.fori_loop(0, 128 // _L, pbody, 0)

    def patch_dma(t, b, slot):
        v8 = pl.multiple_of(jnp.bitwise_and(t, -8), 8)
        bcol = pl.multiple_of(jnp.bitwise_and(b, -128), 128)
        return pltpu.make_async_copy(
            patch.at[slot],
            out_hbm.at[pl.ds(v8, 8), pl.ds(bcol, 128)], sem_p)

    def precompute(t, b, n):
        @pl.when(n < _NSLOT)
        def _():
            compute_patch(t, b, n)

    n_owned = scan(precompute)

    _for_units(wid, out_hbm, tmpl, first, sem_u, lambda cp: cp.start(),
               lo=8, hi=_UPW)
    _for_units(wid, out_hbm, tmpl, first, sem_u, lambda cp: cp.wait())

    def fire(t, b, n):
        @pl.when(n < _NSLOT)
        def _():
            patch_dma(t, b, n).start()

    scan(fire)

    def drain_one():
        pltpu.make_async_copy(
            out_hbm.at[pl.ds(0, 8), pl.ds(0, 128)], patch.at[0], sem_p).wait()

    lax.fori_loop(0, jnp.minimum(n_owned, _NSLOT),
                  lambda i, cc: (drain_one(), cc)[1], 0)

    @pl.when(n_owned > _NSLOT)
    def _():
        def slow(t, b, n):
            @pl.when(n >= _NSLOT)
            def _():
                slot = jnp.bitwise_and(n, 7)
                compute_patch(t, b, slot)
                patch_dma(t, b, slot).start()
                drain_one()

        scan(slow)


def kernel(trg_token_ids_batch):
    trg = trg_token_ids_batch.reshape(_B)
    run = functools.partial(
        pl.kernel,
        out_type=jax.ShapeDtypeStruct((_V, _B), jnp.float32),
        mesh=plsc.VectorSubcoreMesh(core_axis_name="c", subcore_axis_name="s"),
        scratch_types=[
            pltpu.VMEM((_BR, _B), jnp.float32),
            pltpu.VMEM((8, _B), jnp.float32),
            pltpu.VMEM((_NSLOT, 8, 128), jnp.float32),
            pltpu.VMEM((_B,), jnp.int32),
            pltpu.SemaphoreType.DMA,
            pltpu.SemaphoreType.DMA,
        ],
    )(_sc_body)
    return run(trg).T


# speedup vs baseline: 1.0645x; 1.0645x over previous
import functools

import jax
import jax.numpy as jnp
from jax import lax
from jax.experimental import pallas as pl
from jax.experimental.pallas import tpu as pltpu
from jax.experimental.pallas import tpu_sc as plsc

_V = 100000
_B = 1024
_EPS = 0.1 / (_V - 2)
_CONF = 0.9
_NW = 32
_BPW = _B // _NW
_L = 16
_BR = 64
_NUNITS = _V // _BR
_TAIL = _V - _NUNITS * _BR
_UPW = (_NUNITS + 1 + _NW - 1) // _NW


def _lanes():
    return lax.broadcasted_iota(jnp.int32, (_L,), 0)


def _unit_copies(out_hbm, u, tmpl, first, sem):
    off = pl.multiple_of(u * _BR, _BR)
    c_first = [
        pltpu.make_async_copy(first, out_hbm.at[pl.ds(0, 8)], sem),
        pltpu.make_async_copy(
            tmpl.at[pl.ds(0, _BR - 8)], out_hbm.at[pl.ds(8, _BR - 8)], sem),
    ]
    c_mid = [pltpu.make_async_copy(tmpl, out_hbm.at[pl.ds(off, _BR)], sem)]
    c_tail = [pltpu.make_async_copy(
        tmpl.at[pl.ds(0, _TAIL)],
        out_hbm.at[pl.ds(pl.multiple_of(_NUNITS * _BR + 0 * u, 8), _TAIL)],
        sem)]
    return c_first, c_mid, c_tail


def _for_units(wid, out_hbm, tmpl, first, sem, action):
    def body(k, carry):
        u = wid + k * _NW
        c_first, c_mid, c_tail = _unit_copies(out_hbm, u, tmpl, first, sem)

        @pl.when(u == 0)
        def _():
            for cp in c_first:
                action(cp)

        @pl.when((u > 0) & (u < _NUNITS))
        def _():
            for cp in c_mid:
                action(cp)

        @pl.when(u == _NUNITS)
        def _():
            for cp in c_tail:
                action(cp)

        return carry

    lax.fori_loop(0, _UPW, body, 0)


def _sc_body(trg_hbm, out_hbm, tmpl, first, patch, trg_v, sem_u, sem_p):
    wid = lax.axis_index("s") * 2 + lax.axis_index("c")

    pltpu.sync_copy(trg_hbm, trg_v)

    lanes = _lanes()
    zeros = jnp.zeros((_L,), jnp.float32)

    def tbody(c, carry):
        cb = pl.multiple_of(c * _L, _L)
        tv = trg_v[pl.ds(cb, _L)]
        vec = jnp.where(tv == 0, 0.0, _EPS).astype(jnp.float32)
        for i in range(_BR):
            tmpl[i, pl.ds(cb, _L)] = vec
        first[0, pl.ds(cb, _L)] = zeros
        for i in range(1, 8):
            first[i, pl.ds(cb, _L)] = vec
        return carry

    lax.fori_loop(0, _B // _L, tbody, 0)

    _for_units(wid, out_hbm, tmpl, first, sem_u, lambda cp: cp.start())
    _for_units(wid, out_hbm, tmpl, first, sem_u, lambda cp: cp.wait())

    def drain_one():
        pltpu.make_async_copy(
            out_hbm.at[pl.ds(0, 8), pl.ds(0, 128)], patch.at[0], sem_p).wait()

    def scan_body(c, n):
        vec = trg_v[pl.ds(c * _L, _L)]
        mi = jnp.where(
            (vec != 0) & (jnp.bitwise_and(
                lax.shift_right_logical(vec, 6), _NW - 1) == wid),
            1, 0)
        for l in range(_L):
            cond = mi[l] != 0

            @pl.when(cond)
            def _(n=n, c=c, l=l, vec=vec):
                t = vec[l]
                @pl.when((jnp.bitwise_and(n, 7) == 0) & (n > 0))
                def _():
                    lax.fori_loop(0, 8, lambda i, cc: (drain_one(), cc)[1], 0)

                slot = jnp.bitwise_and(n, 7)
                v8 = pl.multiple_of(jnp.bitwise_and(t, -8), 8)
                bcol = pl.multiple_of(
                    jnp.bitwise_and(c * _L + l, -128), 128)

                def pbody(cc, carry):
                    tw = trg_v[pl.ds(bcol + cc * _L, _L)]
                    base = jnp.where(tw == 0, 0.0, _EPS).astype(jnp.float32)
                    for i in range(8):
                        v = v8 + i
                        val = jnp.where(tw == v, _CONF, base)
                        val = jnp.where(v == 0, 0.0, val)
                        patch[slot, i, pl.ds(cc * _L, _L)] = val
                    return carry

                lax.fori_loop(0, 128 // _L, pbody, 0)
                pltpu.make_async_copy(
                    patch.at[slot],
                    out_hbm.at[pl.ds(v8, 8), pl.ds(bcol, 128)], sem_p).start()

            n = jnp.where(cond, n + 1, n)
        return n

    n_fired = lax.fori_loop(0, _B // _L, scan_body, jnp.int32(0))

    @pl.when(n_fired > 0)
    def _():
        rem = jnp.bitwise_and(n_fired - 1, 7) + 1
        lax.fori_loop(0, rem, lambda i, cc: (drain_one(), cc)[1], 0)


def kernel(trg_token_ids_batch):
    trg = trg_token_ids_batch.reshape(_B)
    run = functools.partial(
        pl.kernel,
        out_type=jax.ShapeDtypeStruct((_V, _B), jnp.float32),
        mesh=plsc.VectorSubcoreMesh(core_axis_name="c", subcore_axis_name="s"),
        scratch_types=[
            pltpu.VMEM((_BR, _B), jnp.float32),
            pltpu.VMEM((8, _B), jnp.float32),
            pltpu.VMEM((8, 8, 128), jnp.float32),
            pltpu.VMEM((_B,), jnp.int32),
            pltpu.SemaphoreType.DMA,
            pltpu.SemaphoreType.DMA,
        ],
    )(_sc_body)
    return run(trg).T
